# baseline (speedup 1.0000x reference)
import functools
import jax
import jax.numpy as jnp
from jax import lax
from jax.experimental import pallas as pl
from jax.experimental.pallas import tpu as pltpu, tpu_sc as plsc


def _make_sc(B, R, D, dtype):
    info = plsc.get_sparse_core_info()
    NC, NS = info.num_cores, info.num_subcores
    NW = NC * NS
    rows_per_w = R // NW
    CH = 56
    chunks = []
    off = 0
    while off < rows_per_w:
        ch = min(CH, rows_per_w - off)
        chunks.append((off, ch))
        off += ch

    mesh = plsc.VectorSubcoreMesh(core_axis_name="c", subcore_axis_name="s")

    @functools.partial(
        pl.kernel,
        mesh=mesh,
        out_type=jax.ShapeDtypeStruct((B, R, D), dtype),
        scratch_types=[
            pltpu.VMEM((CH, D), dtype),
            pltpu.VMEM((CH, D), dtype),
            pltpu.SemaphoreType.DMA,
            pltpu.SemaphoreType.DMA,
            pltpu.SemaphoreType.DMA,
            pltpu.SemaphoreType.DMA,
        ],
    )
    def k(wpe_hbm, out_hbm, buf0, buf1, rs0, rs1, ws0, ws1):
        wid = lax.axis_index("s") * NC + lax.axis_index("c")
        base = wid * rows_per_w
        bufs = (buf0, buf1)
        rsems = (rs0, rs1)
        wsems = (ws0, ws1)

        reads = [None, None]
        writes = [[], []]
        o0, c0 = chunks[0]
        reads[0] = pltpu.async_copy(
            wpe_hbm.at[pl.ds(base + o0, c0)], bufs[0].at[pl.ds(0, c0)], rsems[0]
        )
        for kk, (off_k, ch_k) in enumerate(chunks):
            cur = kk % 2
            nxt = (kk + 1) % 2
            if kk + 1 < len(chunks):
                offn, chn = chunks[kk + 1]
                for w in writes[nxt]:
                    w.wait()
                writes[nxt] = []
                reads[nxt] = pltpu.async_copy(
                    wpe_hbm.at[pl.ds(base + offn, chn)],
                    bufs[nxt].at[pl.ds(0, chn)],
                    rsems[nxt],
                )
            reads[cur].wait()
            r0 = base + off_k
            writes[cur] = [
                pltpu.async_copy(
                    bufs[cur].at[pl.ds(0, ch_k)],
                    out_hbm.at[b, pl.ds(r0, ch_k)],
                    wsems[cur],
                )
                for b in range(B)
            ]
        for side in writes:
            for w in side:
                w.wait()

    return k


def kernel(x, wpe):
    B, S = x.shape
    R, D = wpe.shape
    return _make_sc(B, R, D, wpe.dtype)(wpe)


# speedup vs baseline: 1.0226x; 1.0226x over previous
import functools
import jax
import jax.numpy as jnp
from jax import lax
from jax.experimental import pallas as pl
from jax.experimental.pallas import tpu as pltpu, tpu_sc as plsc


def _make_sc(B, R, D, dtype):
    info = plsc.get_sparse_core_info()
    NC, NS = info.num_cores, info.num_subcores
    NW = NC * NS
    rows_per_w = R // NW
    CH = 96
    chunks = []
    off = 0
    while off < rows_per_w:
        ch = min(CH, rows_per_w - off)
        chunks.append((off, ch))
        off += ch

    mesh = plsc.VectorSubcoreMesh(core_axis_name="c", subcore_axis_name="s")

    @functools.partial(
        pl.kernel,
        mesh=mesh,
        out_type=jax.ShapeDtypeStruct((B, R, D), dtype),
        scratch_types=[pltpu.VMEM((CH, D), dtype)],
    )
    def k(wpe_hbm, out_hbm, buf):
        wid = lax.axis_index("s") * NC + lax.axis_index("c")
        base = wid * rows_per_w
        for off_k, ch_k in chunks:
            r0 = base + off_k
            pltpu.sync_copy(wpe_hbm.at[pl.ds(r0, ch_k)], buf.at[pl.ds(0, ch_k)])
            for b in range(B):
                pltpu.sync_copy(
                    buf.at[pl.ds(0, ch_k)], out_hbm.at[b, pl.ds(r0, ch_k)]
                )

    return k


def kernel(x, wpe):
    B, S = x.shape
    R, D = wpe.shape
    return _make_sc(B, R, D, wpe.dtype)(wpe)
